# baseline (speedup 1.0000x reference)
import functools

import jax
import jax.numpy as jnp
from jax.experimental import pallas as pl

K = 1024
D = 64
BETA = 0.25
HW = 1024
B = 16
TILE = 256
N_TILES = HW // TILE


def _vq_kernel(x_ref, emb_ref, q_ref, idx_ref, loss_ref):
    b = pl.program_id(0)
    t = pl.program_id(1)
    x = x_ref[0]
    emb = emb_ref[...]

    e2 = jnp.sum(emb * emb, axis=1, keepdims=True)
    x2 = jnp.sum(x * x, axis=0, keepdims=True)
    m = jax.lax.dot_general(
        emb, x, (((1,), (0,)), ((), ())),
        preferred_element_type=jnp.float32,
    )
    dist = (x2 + e2) - 2.0 * m

    minval = jnp.min(dist, axis=0, keepdims=True)
    kiota = jax.lax.broadcasted_iota(jnp.int32, (K, TILE), 0)
    idx = jnp.min(jnp.where(dist == minval, kiota, K), axis=0)

    onehot = (kiota == idx[None, :]).astype(jnp.float32)
    quant = jax.lax.dot_general(
        emb, onehot, (((0,), (0,)), ((), ())),
        preferred_element_type=jnp.float32,
    )

    q_ref[0] = quant
    idx_ref[0] = idx.reshape(1, TILE)

    partial = jnp.sum(minval).reshape(1, 1)

    @pl.when(jnp.logical_and(b == 0, t == 0))
    def _init():
        loss_ref[...] = partial

    @pl.when(jnp.logical_not(jnp.logical_and(b == 0, t == 0)))
    def _acc():
        loss_ref[...] += partial


@functools.partial(jax.jit)
def kernel(enc_pred, embeddings):
    x3 = enc_pred.reshape(B, D, HW)
    q, idx, loss_raw = pl.pallas_call(
        _vq_kernel,
        grid=(B, N_TILES),
        in_specs=[
            pl.BlockSpec((1, D, TILE), lambda b, t: (b, 0, t)),
            pl.BlockSpec((K, D), lambda b, t: (0, 0)),
        ],
        out_specs=[
            pl.BlockSpec((1, D, TILE), lambda b, t: (b, 0, t)),
            pl.BlockSpec((1, 1, TILE), lambda b, t: (b, 0, t)),
            pl.BlockSpec((1, 1), lambda b, t: (0, 0)),
        ],
        out_shape=[
            jax.ShapeDtypeStruct((B, D, HW), jnp.float32),
            jax.ShapeDtypeStruct((B, 1, HW), jnp.int32),
            jax.ShapeDtypeStruct((1, 1), jnp.float32),
        ],
    )(x3, embeddings)
    quantized_out = q.reshape(B, D, 32, 32)
    indices_out = idx.reshape(B, 1, 32, 32)
    loss = loss_raw[0, 0] * (BETA / (B * HW * D))
    return (quantized_out, loss, indices_out)


# speedup vs baseline: 1.0407x; 1.0407x over previous
import functools

import jax
import jax.numpy as jnp
from jax.experimental import pallas as pl

K = 1024
D = 64
BETA = 0.25
HW = 1024
B = 16
TILE = 256
N_TILES = HW // TILE


def _vq_kernel(x_ref, emb_ref, q_ref, idx_ref, loss_ref):
    b = pl.program_id(0)
    t = pl.program_id(1)
    x = x_ref[0]
    emb = emb_ref[...]

    e2 = jnp.sum(emb * emb, axis=1, keepdims=True)
    x2 = jnp.sum(x * x, axis=0, keepdims=True)
    m = jax.lax.dot_general(
        emb, x, (((1,), (0,)), ((), ())),
        preferred_element_type=jnp.float32,
    )
    dist = (x2 + e2) - 2.0 * m

    idx = jnp.argmin(dist, axis=0)
    kiota = jax.lax.broadcasted_iota(jnp.int32, (K, TILE), 0)
    onehot = (kiota == idx[None, :]).astype(jnp.float32)
    quant = jax.lax.dot_general(
        emb, onehot, (((0,), (0,)), ((), ())),
        preferred_element_type=jnp.float32,
    )

    q_ref[0] = quant
    idx_ref[0] = idx.reshape(1, TILE)

    diff = quant - x
    partial = jnp.sum(diff * diff).reshape(1, 1)

    @pl.when(jnp.logical_and(b == 0, t == 0))
    def _init():
        loss_ref[...] = partial

    @pl.when(jnp.logical_not(jnp.logical_and(b == 0, t == 0)))
    def _acc():
        loss_ref[...] += partial


@functools.partial(jax.jit)
def kernel(enc_pred, embeddings):
    x3 = enc_pred.reshape(B, D, HW)
    q, idx, loss_raw = pl.pallas_call(
        _vq_kernel,
        grid=(B, N_TILES),
        in_specs=[
            pl.BlockSpec((1, D, TILE), lambda b, t: (b, 0, t)),
            pl.BlockSpec((K, D), lambda b, t: (0, 0)),
        ],
        out_specs=[
            pl.BlockSpec((1, D, TILE), lambda b, t: (b, 0, t)),
            pl.BlockSpec((1, 1, TILE), lambda b, t: (b, 0, t)),
            pl.BlockSpec((1, 1), lambda b, t: (0, 0)),
        ],
        out_shape=[
            jax.ShapeDtypeStruct((B, D, HW), jnp.float32),
            jax.ShapeDtypeStruct((B, 1, HW), jnp.int32),
            jax.ShapeDtypeStruct((1, 1), jnp.float32),
        ],
    )(x3, embeddings)
    quantized_out = q.reshape(B, D, 32, 32)
    indices_out = idx.reshape(B, 1, 32, 32)
    loss = loss_raw[0, 0] * (BETA / (B * HW * D))
    return (quantized_out, loss, indices_out)


# speedup vs baseline: 1.6370x; 1.5729x over previous
import jax
import jax.numpy as jnp
from jax.experimental import pallas as pl
from jax.experimental.pallas import tpu as pltpu

K = 1024
D = 64
BETA = 0.25
HW = 1024
B = 16
TILE = 1024
N_TILES = HW // TILE


def _vq_kernel(x_ref, emb_ref, q_ref, idx_ref, loss_ref, e2_ref):
    step = pl.program_id(0)
    x = x_ref[0]
    emb = emb_ref[...]

    @pl.when(step == 0)
    def _prep():
        e2_ref[...] = jnp.sum(emb * emb, axis=1, keepdims=True)

    e2 = e2_ref[...]
    x2 = jnp.sum(x * x, axis=0, keepdims=True)
    m = jax.lax.dot_general(
        emb, x, (((1,), (0,)), ((), ())),
        preferred_element_type=jnp.float32,
    )
    dist = (x2 + e2) - 2.0 * m

    idx = jnp.argmin(dist, axis=0)
    kiota = jax.lax.broadcasted_iota(jnp.int32, (K, TILE), 0)
    onehot = (kiota == idx[None, :]).astype(jnp.float32)
    quant = jax.lax.dot_general(
        emb, onehot, (((0,), (0,)), ((), ())),
        preferred_element_type=jnp.float32,
    )

    q_ref[0] = quant
    idx_ref[0] = idx.reshape(1, TILE)

    diff = quant - x
    partial = jnp.sum(diff * diff).reshape(1, 1)

    @pl.when(step == 0)
    def _init():
        loss_ref[...] = partial

    @pl.when(step != 0)
    def _acc():
        loss_ref[...] += partial


@jax.jit
def kernel(enc_pred, embeddings):
    x3 = enc_pred.reshape(B, D, HW)
    q, idx, loss_raw = pl.pallas_call(
        _vq_kernel,
        grid=(B * N_TILES,),
        in_specs=[
            pl.BlockSpec((1, D, TILE), lambda s: (s, 0, 0)),
            pl.BlockSpec((K, D), lambda s: (0, 0)),
        ],
        out_specs=[
            pl.BlockSpec((1, D, TILE), lambda s: (s, 0, 0)),
            pl.BlockSpec((1, 1, TILE), lambda s: (s, 0, 0)),
            pl.BlockSpec((1, 1), lambda s: (0, 0)),
        ],
        out_shape=[
            jax.ShapeDtypeStruct((B * N_TILES, D, TILE), jnp.float32),
            jax.ShapeDtypeStruct((B * N_TILES, 1, TILE), jnp.int32),
            jax.ShapeDtypeStruct((1, 1), jnp.float32),
        ],
        scratch_shapes=[pltpu.VMEM((K, 1), jnp.float32)],
    )(x3, embeddings)
    quantized_out = q.reshape(B, D, 32, 32)
    indices_out = idx.reshape(B, 1, 32, 32)
    loss = loss_raw[0, 0] * (BETA / (B * HW * D))
    return (quantized_out, loss, indices_out)
